# baseline (speedup 1.0000x reference)
import functools

import jax
import jax.numpy as jnp
from jax import lax
from jax.experimental import pallas as pl
from jax.experimental.pallas import tpu as pltpu
from jax.experimental.pallas import tpu_sc as plsc

_B, _N, _D = 4, 8192, 768
_NW = 32
_ROWS_PER_W = _N // _NW
_R = 16
_CH = _R * _D
_STEPS = _ROWS_PER_W // _R


def _sc_body(enc_hbm, tab_hbm, out_hbm, tab_v, *bufs):
    enc_v, out_v = bufs[0:4], bufs[4:8]
    enc_s, out_s = bufs[8:12], bufs[12:16]
    wid = lax.axis_index("s") * 2 + lax.axis_index("c")
    wbase = wid * _ROWS_PER_W

    def off(t, b):
        return b * (_N * _D) + (wbase + t * _R) * _D

    for p in range(_B):
        pltpu.make_async_copy(
            enc_hbm.at[pl.ds(off(0, p), _CH)], enc_v[p], enc_s[p]).start()

    def step(g, carry):
        pltpu.sync_copy(tab_hbm.at[pl.ds((wbase + g * _R) * _D, _CH)], tab_v)

        for p in range(_B):
            o = off(g, p)

            pltpu.make_async_copy(
                enc_hbm.at[pl.ds(o, _CH)], enc_v[p], enc_s[p]).wait()

            @pl.when(g >= 1)
            def _():
                pltpu.make_async_copy(
                    out_v[p], out_hbm.at[pl.ds(o, _CH)], out_s[p]).wait()

            ev, ov = enc_v[p], out_v[p]

            @plsc.parallel_loop(0, _CH // 16, unroll=8)
            def _(j):
                sl = pl.ds(j * 16, 16)
                ov[sl] = ev[sl] + tab_v[sl]

            pltpu.make_async_copy(
                out_v[p], out_hbm.at[pl.ds(o, _CH)], out_s[p]).start()

            @pl.when(g < _STEPS - 1)
            def _():
                pltpu.make_async_copy(
                    enc_hbm.at[pl.ds(off(g + 1, p), _CH)],
                    enc_v[p], enc_s[p]).start()

        return carry

    lax.fori_loop(0, _STEPS, step, 0)

    for p in range(_B):
        pltpu.make_async_copy(
            out_v[p], out_hbm.at[pl.ds(0, _CH)], out_s[p]).wait()


_sc_kernel = functools.partial(
    pl.kernel,
    mesh=plsc.VectorSubcoreMesh(core_axis_name="c", subcore_axis_name="s"),
    out_type=jax.ShapeDtypeStruct((_B * _N * _D,), jnp.float32),
    scratch_types=(
        [pltpu.VMEM((_CH,), jnp.float32)] * 9
        + [pltpu.SemaphoreType.DMA] * 8
    ),
)(_sc_body)


def kernel(encoded_tokens, pos_table):
    b, n, d = encoded_tokens.shape
    flat = _sc_kernel(encoded_tokens.reshape(-1), pos_table.reshape(-1))
    return flat.reshape(b, n, d)


# speedup vs baseline: 4.7406x; 4.7406x over previous
import jax
import jax.numpy as jnp
from jax.experimental import pallas as pl


_BN = 512


def _add_kernel(enc_ref, pos_ref, out_ref):
    out_ref[...] = enc_ref[...] + pos_ref[...]


def kernel(encoded_tokens, pos_table):
    b, n, d = encoded_tokens.shape
    num_n = n // _BN
    return pl.pallas_call(
        _add_kernel,
        grid=(num_n,),
        in_specs=[
            pl.BlockSpec((b, _BN, d), lambda i: (0, i, 0)),
            pl.BlockSpec((1, _BN, d), lambda i: (0, i, 0)),
        ],
        out_specs=pl.BlockSpec((b, _BN, d), lambda i: (0, i, 0)),
        out_shape=jax.ShapeDtypeStruct((b, n, d), encoded_tokens.dtype),
    )(encoded_tokens, pos_table[None])
